# baseline (speedup 1.0000x reference)
import functools

import jax
import jax.numpy as jnp
from jax import lax
from jax.experimental import pallas as pl
from jax.experimental.pallas import tpu as pltpu
from jax.experimental.pallas import tpu_sc as plsc

N_NODES = 1_000_000
CHUNK = 8_000
N_CHUNKS = N_NODES // CHUNK
N_WORKERS = 32
MAX_ITERS = -(-N_CHUNKS // N_WORKERS)
LANES = 16


def _sc_stream_body(fringe_hbm, minf_hbm, term_hbm, stat_hbm,
                    out_hbm, part_hbm,
                    f_v, m_v, t_v, s_v, o_v, acc_v):
    wid = lax.axis_index("s") * 2 + lax.axis_index("c")
    acc_v[...] = jnp.zeros((LANES,), jnp.float32)

    for it in range(MAX_ITERS):
        chunk = wid + it * N_WORKERS

        @pl.when(chunk < N_CHUNKS)
        def _process():
            off = chunk * CHUNK
            pltpu.sync_copy(fringe_hbm.at[pl.ds(off, CHUNK)], f_v)
            pltpu.sync_copy(minf_hbm.at[pl.ds(off, CHUNK)], m_v)
            pltpu.sync_copy(term_hbm.at[pl.ds(off, CHUNK)], t_v)
            pltpu.sync_copy(stat_hbm.at[pl.ds(off, CHUNK)], s_v)

            def body(j, acc):
                sl = pl.ds(j * LANES, LANES)
                f = f_v[sl]
                o_v[sl] = jnp.where(s_v[sl] == 0, f, m_v[sl])
                return acc + f * t_v[sl].astype(jnp.float32)

            acc = lax.fori_loop(0, CHUNK // LANES, body, acc_v[...])
            acc_v[...] = acc
            pltpu.sync_copy(o_v, out_hbm.at[pl.ds(off, CHUNK)])

    pltpu.sync_copy(acc_v, part_hbm.at[wid])


@functools.partial(
    pl.kernel,
    out_type=(
        jax.ShapeDtypeStruct((N_NODES,), jnp.float32),
        jax.ShapeDtypeStruct((N_WORKERS, LANES), jnp.float32),
    ),
    mesh=plsc.VectorSubcoreMesh(core_axis_name="c", subcore_axis_name="s"),
    scratch_types=[
        pltpu.VMEM((CHUNK,), jnp.float32),
        pltpu.VMEM((CHUNK,), jnp.float32),
        pltpu.VMEM((CHUNK,), jnp.int32),
        pltpu.VMEM((CHUNK,), jnp.int32),
        pltpu.VMEM((CHUNK,), jnp.float32),
        pltpu.VMEM((LANES,), jnp.float32),
    ],
)
def _sc_stream(*args):
    _sc_stream_body(*args)


def _flux_body(step_ref, part_ref, flux_ref, out_ref):
    total = jnp.sum(part_ref[...])
    step = step_ref[0, 0]
    rows = lax.broadcasted_iota(jnp.int32, (8, 125), 0)
    cols = lax.broadcasted_iota(jnp.int32, (8, 125), 1)
    flat_idx = rows * 125 + cols
    out_ref[...] = jnp.where(flat_idx == step, total, flux_ref[...])


def _flux_update(step2d, partials, flux2d):
    return pl.pallas_call(
        _flux_body,
        out_shape=jax.ShapeDtypeStruct((8, 125), jnp.float32),
        in_specs=[
            pl.BlockSpec(memory_space=pltpu.SMEM),
            pl.BlockSpec(memory_space=pltpu.VMEM),
            pl.BlockSpec(memory_space=pltpu.VMEM),
        ],
        out_specs=pl.BlockSpec(memory_space=pltpu.VMEM),
    )(step2d, partials, flux2d)


def kernel(fringe_thickness, min_fringe_thickness, fluxes, node_is_terminus,
           status_at_node, current_step):
    cleared, partials = _sc_stream(fringe_thickness, min_fringe_thickness,
                                   node_is_terminus, status_at_node)
    step2d = jnp.asarray(current_step, jnp.int32).reshape(1, 1)
    flux2d = fluxes.reshape(8, 125)
    out2d = _flux_update(step2d, partials, flux2d)
    return cleared, out2d.reshape(fluxes.shape)


# speedup vs baseline: 1.2333x; 1.2333x over previous
import functools

import jax
import jax.numpy as jnp
from jax import lax
from jax.experimental import pallas as pl
from jax.experimental.pallas import tpu as pltpu
from jax.experimental.pallas import tpu_sc as plsc

N_NODES = 1_000_000
CHUNK = 8_000
N_CHUNKS = N_NODES // CHUNK
N_WORKERS = 32
MAX_ITERS = -(-N_CHUNKS // N_WORKERS)
LANES = 16
GROUPS = 4


def _sc_stream_body(fringe_hbm, minf_hbm, term_hbm, stat_hbm,
                    out_hbm, part_hbm,
                    f0, f1, m0, m1, t0, t1, s0, s1, o0, o1, acc_v,
                    isem0, isem1, osem0, osem1):
    wid = lax.axis_index("s") * 2 + lax.axis_index("c")
    f_v, m_v, t_v, s_v, o_v = (f0, f1), (m0, m1), (t0, t1), (s0, s1), (o0, o1)
    isems = (isem0, isem1)
    osems = (osem0, osem1)
    acc_v[...] = jnp.zeros((LANES,), jnp.float32)

    def in_copies(slot, chunk):
        off = chunk * CHUNK
        sl = pl.ds(off, CHUNK)
        return (
            pltpu.make_async_copy(fringe_hbm.at[sl], f_v[slot], isems[slot]),
            pltpu.make_async_copy(minf_hbm.at[sl], m_v[slot], isems[slot]),
            pltpu.make_async_copy(term_hbm.at[sl], t_v[slot], isems[slot]),
            pltpu.make_async_copy(stat_hbm.at[sl], s_v[slot], isems[slot]),
        )

    def out_copy(slot, chunk):
        off = chunk * CHUNK
        return pltpu.make_async_copy(
            o_v[slot], out_hbm.at[pl.ds(off, CHUNK)], osems[slot])

    for it in range(MAX_ITERS):
        chunk = wid + it * N_WORKERS
        slot = it % 2

        if it == 0:
            @pl.when(chunk < N_CHUNKS)
            def _prime():
                for c in in_copies(0, chunk):
                    c.start()

        if it + 1 < MAX_ITERS:
            nxt = chunk + N_WORKERS

            @pl.when(nxt < N_CHUNKS)
            def _prefetch():
                for c in in_copies(1 - slot, nxt):
                    c.start()

        @pl.when(chunk < N_CHUNKS)
        def _process():
            for c in in_copies(slot, chunk):
                c.wait()
            if it >= 2:
                out_copy(slot, chunk - 2 * N_WORKERS).wait()

            zero = jnp.zeros((LANES,), jnp.float32)

            @plsc.parallel_loop(0, CHUNK, step=GROUPS * LANES, unroll=2,
                                carry=(zero, zero, zero, zero))
            def body(j, accs):
                new = []
                for g in range(GROUPS):
                    sl = pl.ds(j + g * LANES, LANES)
                    f = f_v[slot][sl]
                    o_v[slot][sl] = jnp.where(s_v[slot][sl] == 0, f, m_v[slot][sl])
                    new.append(accs[g] + f * t_v[slot][sl].astype(jnp.float32))
                return tuple(new)

            a0, a1, a2, a3 = body
            acc_v[...] = acc_v[...] + ((a0 + a1) + (a2 + a3))
            out_copy(slot, chunk).start()

    for it in range(MAX_ITERS):
        chunk = wid + it * N_WORKERS

        @pl.when((chunk < N_CHUNKS) & (chunk + 2 * N_WORKERS >= N_CHUNKS))
        def _drain():
            out_copy(it % 2, chunk).wait()

    pltpu.sync_copy(acc_v, part_hbm.at[wid])


@functools.partial(
    pl.kernel,
    out_type=(
        jax.ShapeDtypeStruct((N_NODES,), jnp.float32),
        jax.ShapeDtypeStruct((N_WORKERS, LANES), jnp.float32),
    ),
    mesh=plsc.VectorSubcoreMesh(core_axis_name="c", subcore_axis_name="s"),
    scratch_types=[
        pltpu.VMEM((CHUNK,), jnp.float32),
        pltpu.VMEM((CHUNK,), jnp.float32),
        pltpu.VMEM((CHUNK,), jnp.float32),
        pltpu.VMEM((CHUNK,), jnp.float32),
        pltpu.VMEM((CHUNK,), jnp.int32),
        pltpu.VMEM((CHUNK,), jnp.int32),
        pltpu.VMEM((CHUNK,), jnp.int32),
        pltpu.VMEM((CHUNK,), jnp.int32),
        pltpu.VMEM((CHUNK,), jnp.float32),
        pltpu.VMEM((CHUNK,), jnp.float32),
        pltpu.VMEM((LANES,), jnp.float32),
        pltpu.SemaphoreType.DMA,
        pltpu.SemaphoreType.DMA,
        pltpu.SemaphoreType.DMA,
        pltpu.SemaphoreType.DMA,
    ],
)
def _sc_stream(*args):
    _sc_stream_body(*args)


def _flux_body(step_ref, part_ref, flux_ref, out_ref):
    total = jnp.sum(part_ref[...])
    step = step_ref[0, 0]
    rows = lax.broadcasted_iota(jnp.int32, (8, 125), 0)
    cols = lax.broadcasted_iota(jnp.int32, (8, 125), 1)
    flat_idx = rows * 125 + cols
    out_ref[...] = jnp.where(flat_idx == step, total, flux_ref[...])


def _flux_update(step2d, partials, flux2d):
    return pl.pallas_call(
        _flux_body,
        out_shape=jax.ShapeDtypeStruct((8, 125), jnp.float32),
        in_specs=[
            pl.BlockSpec(memory_space=pltpu.SMEM),
            pl.BlockSpec(memory_space=pltpu.VMEM),
            pl.BlockSpec(memory_space=pltpu.VMEM),
        ],
        out_specs=pl.BlockSpec(memory_space=pltpu.VMEM),
    )(step2d, partials, flux2d)


def kernel(fringe_thickness, min_fringe_thickness, fluxes, node_is_terminus,
           status_at_node, current_step):
    cleared, partials = _sc_stream(fringe_thickness, min_fringe_thickness,
                                   node_is_terminus, status_at_node)
    step2d = jnp.asarray(current_step, jnp.int32).reshape(1, 1)
    flux2d = fluxes.reshape(8, 125)
    out2d = _flux_update(step2d, partials, flux2d)
    return cleared, out2d.reshape(fluxes.shape)
